# baseline (speedup 1.0000x reference)
import functools

import jax
import jax.numpy as jnp
import numpy as np
from jax import lax
from jax.experimental import pallas as pl
from jax.experimental.pallas import tpu as pltpu
from jax.experimental.pallas import tpu_sc as plsc

VSIZE = np.array([0.1, 0.1, 0.15], np.float32)
PMIN = np.array([-51.2, -51.2, -3.0], np.float32)
GRID = np.array([1024, 1024, 40], np.int32)
GRID_TOTAL = int(GRID[0]) * int(GRID[1]) * int(GRID[2])
MAX_VOXELS = 150000
MAX_PTS = 5
NFEAT = 4
BATCH = 2
N_PTS = 400000
PW = 8

NW = 32
VIDS = BATCH * MAX_VOXELS
VIDS_PAD = 307200
V_PER_W = VIDS_PAD // NW
N_OUTER = 5
V_CHUNK = V_PER_W // N_OUTER
R_CHUNK = V_CHUNK * MAX_PTS
G_IDX = 128
N_GRP = R_CHUNK // G_IDX
N_SUB = 5
SUB_GRP = N_GRP // N_SUB
SUB_R = R_CHUNK // N_SUB


def _sc_body(pts_hbm, pidx_hbm, vlen_hbm, vkey_hbm,
             vox_hbm, vc_hbm, np_hbm,
             idx_v, pts_v, vox_v, vl_v, vk_v, vc_v, sem):
    wid = lax.axis_index("s") * 2 + lax.axis_index("c")
    lane = lax.iota(jnp.int32, 16)
    zero_f = jnp.zeros((16,), jnp.float32)
    zero_i = jnp.zeros((16,), jnp.int32)

    def outer(o, _):
        vbase = pl.multiple_of(wid * V_PER_W + o * V_CHUNK, 8)
        rbase4 = pl.multiple_of(vbase * (MAX_PTS * NFEAT), 8)
        ibase = pl.multiple_of((wid * N_OUTER + o) * R_CHUNK, 8)
        pltpu.sync_copy(pidx_hbm.at[pl.ds(ibase, R_CHUNK)], idx_v)
        pltpu.sync_copy(vlen_hbm.at[pl.ds(vbase, V_CHUNK)], vl_v)
        pltpu.sync_copy(vkey_hbm.at[pl.ds(vbase, V_CHUNK)], vk_v)

        def remap(q, _):
            sl = pl.ds(q * 16, 16)
            v = idx_v[sl]
            spread = q * 16 + lane
            idx_v[sl] = jnp.where(v > 0, v - 1, spread)
            return 0

        lax.fori_loop(0, R_CHUNK // 16, remap, 0, unroll=4)

        def sub(s, _):
            def f(i, _):
                g = s * SUB_GRP + i
                pltpu.async_copy(pts_hbm.at[idx_v.at[pl.ds(g * G_IDX, G_IDX)]],
                                 pts_v.at[pl.ds(i * G_IDX, G_IDX)], sem)
                return 0

            lax.fori_loop(0, SUB_GRP, f, 0)

            def d(i, _):
                g = s * SUB_GRP + i
                pltpu.make_async_copy(pts_hbm.at[idx_v.at[pl.ds(g * G_IDX, G_IDX)]],
                                      pts_v.at[pl.ds(i * G_IDX, G_IDX)],
                                      sem).wait()
                return 0

            lax.fori_loop(0, SUB_GRP, d, 0)

            def rows(t, _):
                for k in range(5):
                    rk = k * 16 + lane
                    voff = rk // 5
                    jk = rk - voff * 5
                    v = s * (SUB_R // 5) + t * 16 + voff
                    rloc = t * 80 + rk
                    rglob = s * SUB_R + rloc
                    cnt = plsc.load_gather(vl_v, [v])
                    m = jk < cnt
                    for c in range(NFEAT):
                        col = jnp.full((16,), c + 1, jnp.int32)
                        val = plsc.load_gather(pts_v, [rloc, col])
                        val = jnp.where(m, val, zero_f)
                        plsc.store_scatter(vox_v, [rglob * NFEAT + c], val)
                return 0

            lax.fori_loop(0, SUB_R // 80, rows, 0)
            return 0

        lax.fori_loop(0, N_SUB, sub, 0)

        def vcs(t, _):
            sl = pl.ds(t * 16, 16)
            key = vk_v[sl]
            occ = vl_v[sl] > 0
            pb = key // GRID_TOTAL
            lin = key - pb * GRID_TOTAL
            x = lin & 1023
            y = (lin >> 10) & 1023
            z = lin >> 20
            v = t * 16 + lane
            for c, valc in enumerate((pb, z, y, x)):
                plsc.store_scatter(vc_v, [v * 4 + c],
                                   jnp.where(occ, valc, zero_i))
            return 0

        lax.fori_loop(0, V_CHUNK // 16, vcs, 0)

        pltpu.sync_copy(vox_v, vox_hbm.at[pl.ds(rbase4, R_CHUNK * NFEAT)])
        pltpu.sync_copy(vc_v, vc_hbm.at[pl.ds(vbase * 4, V_CHUNK * 4)])
        pltpu.sync_copy(vl_v, np_hbm.at[pl.ds(vbase, V_CHUNK)])
        return 0

    lax.fori_loop(0, N_OUTER, outer, 0)


@functools.partial(
    pl.kernel,
    out_type=(
        jax.ShapeDtypeStruct((VIDS_PAD * MAX_PTS * NFEAT,), jnp.float32),
        jax.ShapeDtypeStruct((VIDS_PAD * 4,), jnp.int32),
        jax.ShapeDtypeStruct((VIDS_PAD,), jnp.int32),
    ),
    mesh=plsc.VectorSubcoreMesh(core_axis_name="c", subcore_axis_name="s"),
    scratch_types=[
        pltpu.VMEM((R_CHUNK,), jnp.int32),
        pltpu.VMEM((SUB_R, PW), jnp.float32),
        pltpu.VMEM((R_CHUNK * NFEAT,), jnp.float32),
        pltpu.VMEM((V_CHUNK,), jnp.int32),
        pltpu.VMEM((V_CHUNK,), jnp.int32),
        pltpu.VMEM((V_CHUNK * 4,), jnp.int32),
        pltpu.SemaphoreType.DMA,
    ],
    compiler_params=pltpu.CompilerParams(use_tc_tiling_on_sc=False,
                                         needs_layout_passes=False),
)
def _sc_materialize(pts_hbm, pidx_hbm, vlen_hbm, vkey_hbm,
                    vox_hbm, vc_hbm, np_hbm,
                    idx_v, pts_v, vox_v, vl_v, vk_v, vc_v, sem):
    _sc_body(pts_hbm, pidx_hbm, vlen_hbm, vkey_hbm,
             vox_hbm, vc_hbm, np_hbm,
             idx_v, pts_v, vox_v, vl_v, vk_v, vc_v, sem)


def kernel(points):
    n = points.shape[0]
    b = lax.stop_gradient(points[:, 0]).astype(jnp.int32)
    xyz = lax.stop_gradient(points[:, 1:4])
    coords = jnp.floor((xyz - PMIN) / VSIZE).astype(jnp.int32)
    in_range = jnp.all((coords >= 0) & (coords < GRID), axis=1)
    lin = (coords[:, 2] * (int(GRID[1]) * int(GRID[0]))
           + coords[:, 1] * int(GRID[0]) + coords[:, 0])
    sentinel = BATCH * GRID_TOTAL
    key = jnp.where(in_range, b * GRID_TOTAL + lin, sentinel)

    iota = jnp.arange(n, dtype=jnp.int32)
    skey, order = lax.sort((key, iota), num_keys=1, is_stable=True)

    is_new = jnp.concatenate([jnp.ones((1,), bool), skey[1:] != skey[:-1]])
    first = lax.cummax(jnp.where(is_new, iota, -1), axis=0)
    slot = iota - first
    seg = jnp.cumsum(is_new.astype(jnp.int32)) - 1
    nvox0 = jnp.sum((is_new & (skey < GRID_TOTAL)).astype(jnp.int32))
    pb = jnp.clip(skey // GRID_TOTAL, 0, BATCH).astype(jnp.int32)
    rank = seg - jnp.where(pb >= 1, nvox0, 0)
    valid = (skey < sentinel) & (slot < MAX_PTS) & (rank < MAX_VOXELS)
    vid = jnp.where(valid, pb * MAX_VOXELS + rank, VIDS_PAD)
    vstart = valid & (slot == 0)

    TRASH = VIDS_PAD * MAX_PTS
    pidx = jnp.zeros((TRASH + 1,), jnp.int32).at[
        jnp.where(valid, vid * MAX_PTS + slot, TRASH)].add(order + 1)
    pidx = pidx[:TRASH]
    vlen = jnp.zeros((VIDS_PAD + 1,), jnp.int32).at[vid].add(
        valid.astype(jnp.int32))[:VIDS_PAD]
    vkey = jnp.zeros((VIDS_PAD + 1,), jnp.int32).at[
        jnp.where(vstart, vid, VIDS_PAD)].add(skey)[:VIDS_PAD]

    pts8 = jnp.concatenate(
        [points, jnp.zeros((n, PW - points.shape[1]), jnp.float32)], axis=1)
    vox_flat, vc_flat, np_pad = _sc_materialize(pts8, pidx, vlen, vkey)
    voxels = vox_flat[: VIDS * MAX_PTS * NFEAT].reshape(VIDS, MAX_PTS, NFEAT)
    vcoords = vc_flat[: VIDS * 4].reshape(VIDS, 4)
    return voxels, vcoords, np_pad[:VIDS]


# speedup vs baseline: 1.1104x; 1.1104x over previous
import functools

import jax
import jax.numpy as jnp
import numpy as np
from jax import lax
from jax.experimental import pallas as pl
from jax.experimental.pallas import tpu as pltpu
from jax.experimental.pallas import tpu_sc as plsc

VSIZE = np.array([0.1, 0.1, 0.15], np.float32)
PMIN = np.array([-51.2, -51.2, -3.0], np.float32)
GRID = np.array([1024, 1024, 40], np.int32)
GRID_TOTAL = int(GRID[0]) * int(GRID[1]) * int(GRID[2])
MAX_VOXELS = 150000
MAX_PTS = 5
NFEAT = 4
BATCH = 2
N_PTS = 400000
PW = 8

NW = 32
VIDS = BATCH * MAX_VOXELS
VIDS_PAD = 307200
V_PER_W = VIDS_PAD // NW
N_OUTER = 5
V_CHUNK = V_PER_W // N_OUTER
R_CHUNK = V_CHUNK * MAX_PTS
G_IDX = 128
N_GRP = R_CHUNK // G_IDX
N_SUB = 5
SUB_GRP = N_GRP // N_SUB
SUB_R = R_CHUNK // N_SUB

N8 = 400384
ROWS_W = N8 // NW
SUB_A = 4
ROWS_S = ROWS_W // SUB_A
SRCW_S = ROWS_S * 5
SRCW_W = ROWS_W * 5
NFLAT = N8 * 5


@functools.partial(
    pl.kernel,
    out_type=jax.ShapeDtypeStruct((N8, PW), jnp.float32),
    mesh=plsc.VectorSubcoreMesh(core_axis_name="c", subcore_axis_name="s"),
    scratch_types=[
        pltpu.VMEM((SRCW_S,), jnp.float32),
        pltpu.VMEM((ROWS_S, PW), jnp.float32),
    ],
    compiler_params=pltpu.CompilerParams(use_tc_tiling_on_sc=False,
                                         needs_layout_passes=False),
)
def _sc_pad_rows(src_hbm, out_hbm, src_v, dst_v):
    wid = lax.axis_index("s") * 2 + lax.axis_index("c")
    lane = lax.iota(jnp.int32, 16)
    lane8 = lane & 7
    half = lane >> 3
    zf = jnp.zeros((16,), jnp.float32)

    def sub(s, _):
        sbase = pl.multiple_of(wid * SRCW_W + s * SRCW_S, 8)
        dbase = pl.multiple_of(wid * ROWS_W + s * ROWS_S, 8)
        pltpu.sync_copy(src_hbm.at[pl.ds(sbase, SRCW_S)], src_v)

        def cv(t, _):
            r = t * 2 + half
            si = jnp.minimum(r * 5 + lane8, SRCW_S - 1)
            val = plsc.load_gather(src_v, [si])
            val = jnp.where(lane8 < 5, val, zf)
            plsc.store_scatter(dst_v, [r, lane8], val)
            return 0

        lax.fori_loop(0, ROWS_S * PW // 16, cv, 0, unroll=4)
        pltpu.sync_copy(dst_v, out_hbm.at[pl.ds(dbase, ROWS_S)])
        return 0

    lax.fori_loop(0, SUB_A, sub, 0)


def _sc_body(pts_hbm, pidx_hbm, vlen_hbm, vkey_hbm,
             vox_hbm, vc_hbm, np_hbm,
             idx_v, pts_v, vox_v, vl_v, vk_v, vc_v, sem):
    wid = lax.axis_index("s") * 2 + lax.axis_index("c")
    lane = lax.iota(jnp.int32, 16)
    zero_f = jnp.zeros((16,), jnp.float32)
    zero_i = jnp.zeros((16,), jnp.int32)

    def outer(o, _):
        vbase = pl.multiple_of(wid * V_PER_W + o * V_CHUNK, 8)
        rbase4 = pl.multiple_of(vbase * (MAX_PTS * NFEAT), 8)
        ibase = pl.multiple_of((wid * N_OUTER + o) * R_CHUNK, 8)
        pltpu.sync_copy(pidx_hbm.at[pl.ds(ibase, R_CHUNK)], idx_v)
        pltpu.sync_copy(vlen_hbm.at[pl.ds(vbase, V_CHUNK)], vl_v)
        pltpu.sync_copy(vkey_hbm.at[pl.ds(vbase, V_CHUNK)], vk_v)

        def remap(q, _):
            sl = pl.ds(q * 16, 16)
            v = idx_v[sl]
            spread = q * 16 + lane
            idx_v[sl] = jnp.where(v > 0, v - 1, spread)
            return 0

        lax.fori_loop(0, R_CHUNK // 16, remap, 0, unroll=4)

        def sub(s, _):
            def f(i, _):
                g = s * SUB_GRP + i
                pltpu.async_copy(pts_hbm.at[idx_v.at[pl.ds(g * G_IDX, G_IDX)]],
                                 pts_v.at[pl.ds(i * G_IDX, G_IDX)], sem)
                return 0

            lax.fori_loop(0, SUB_GRP, f, 0)

            def d(i, _):
                g = s * SUB_GRP + i
                pltpu.make_async_copy(pts_hbm.at[idx_v.at[pl.ds(g * G_IDX, G_IDX)]],
                                      pts_v.at[pl.ds(i * G_IDX, G_IDX)],
                                      sem).wait()
                return 0

            lax.fori_loop(0, SUB_GRP, d, 0)

            def rows(t, _):
                for k in range(5):
                    rk = k * 16 + lane
                    voff = rk // 5
                    jk = rk - voff * 5
                    v = s * (SUB_R // 5) + t * 16 + voff
                    rloc = t * 80 + rk
                    rglob = s * SUB_R + rloc
                    cnt = plsc.load_gather(vl_v, [v])
                    m = jk < cnt
                    for c in range(NFEAT):
                        col = jnp.full((16,), c + 1, jnp.int32)
                        val = plsc.load_gather(pts_v, [rloc, col])
                        val = jnp.where(m, val, zero_f)
                        plsc.store_scatter(vox_v, [rglob * NFEAT + c], val)
                return 0

            lax.fori_loop(0, SUB_R // 80, rows, 0)
            return 0

        lax.fori_loop(0, N_SUB, sub, 0)

        def vcs(t, _):
            sl = pl.ds(t * 16, 16)
            key = vk_v[sl]
            occ = vl_v[sl] > 0
            pb = key // GRID_TOTAL
            lin = key - pb * GRID_TOTAL
            x = lin & 1023
            y = (lin >> 10) & 1023
            z = lin >> 20
            v = t * 16 + lane
            for c, valc in enumerate((pb, z, y, x)):
                plsc.store_scatter(vc_v, [v * 4 + c],
                                   jnp.where(occ, valc, zero_i))
            return 0

        lax.fori_loop(0, V_CHUNK // 16, vcs, 0)

        pltpu.sync_copy(vox_v, vox_hbm.at[pl.ds(rbase4, R_CHUNK * NFEAT)])
        pltpu.sync_copy(vc_v, vc_hbm.at[pl.ds(vbase * 4, V_CHUNK * 4)])
        pltpu.sync_copy(vl_v, np_hbm.at[pl.ds(vbase, V_CHUNK)])
        return 0

    lax.fori_loop(0, N_OUTER, outer, 0)


@functools.partial(
    pl.kernel,
    out_type=(
        jax.ShapeDtypeStruct((VIDS_PAD * MAX_PTS * NFEAT,), jnp.float32),
        jax.ShapeDtypeStruct((VIDS_PAD * 4,), jnp.int32),
        jax.ShapeDtypeStruct((VIDS_PAD,), jnp.int32),
    ),
    mesh=plsc.VectorSubcoreMesh(core_axis_name="c", subcore_axis_name="s"),
    scratch_types=[
        pltpu.VMEM((R_CHUNK,), jnp.int32),
        pltpu.VMEM((SUB_R, PW), jnp.float32),
        pltpu.VMEM((R_CHUNK * NFEAT,), jnp.float32),
        pltpu.VMEM((V_CHUNK,), jnp.int32),
        pltpu.VMEM((V_CHUNK,), jnp.int32),
        pltpu.VMEM((V_CHUNK * 4,), jnp.int32),
        pltpu.SemaphoreType.DMA,
    ],
    compiler_params=pltpu.CompilerParams(use_tc_tiling_on_sc=False,
                                         needs_layout_passes=False),
)
def _sc_materialize(pts_hbm, pidx_hbm, vlen_hbm, vkey_hbm,
                    vox_hbm, vc_hbm, np_hbm,
                    idx_v, pts_v, vox_v, vl_v, vk_v, vc_v, sem):
    _sc_body(pts_hbm, pidx_hbm, vlen_hbm, vkey_hbm,
             vox_hbm, vc_hbm, np_hbm,
             idx_v, pts_v, vox_v, vl_v, vk_v, vc_v, sem)


def kernel(points):
    n = points.shape[0]
    b = lax.stop_gradient(points[:, 0]).astype(jnp.int32)
    xyz = lax.stop_gradient(points[:, 1:4])
    coords = jnp.floor((xyz - PMIN) / VSIZE).astype(jnp.int32)
    in_range = jnp.all((coords >= 0) & (coords < GRID), axis=1)
    lin = (coords[:, 2] * (int(GRID[1]) * int(GRID[0]))
           + coords[:, 1] * int(GRID[0]) + coords[:, 0])
    sentinel = BATCH * GRID_TOTAL
    key = jnp.where(in_range, b * GRID_TOTAL + lin, sentinel)

    iota = jnp.arange(n, dtype=jnp.int32)
    skey, order = lax.sort((key, iota), num_keys=1, is_stable=True)

    is_new = jnp.concatenate([jnp.ones((1,), bool), skey[1:] != skey[:-1]])
    first = lax.cummax(jnp.where(is_new, iota, -1), axis=0)
    slot = iota - first
    seg = jnp.cumsum(is_new.astype(jnp.int32)) - 1
    nvox0 = jnp.sum((is_new & (skey < GRID_TOTAL)).astype(jnp.int32))
    pb = jnp.clip(skey // GRID_TOTAL, 0, BATCH).astype(jnp.int32)
    rank = seg - jnp.where(pb >= 1, nvox0, 0)
    valid = (skey < sentinel) & (slot < MAX_PTS) & (rank < MAX_VOXELS)
    vid = jnp.where(valid, pb * MAX_VOXELS + rank, VIDS_PAD)
    vstart = valid & (slot == 0)

    TRASH = VIDS_PAD * MAX_PTS
    pidx = jnp.zeros((TRASH + 1,), jnp.int32).at[
        jnp.where(valid, vid * MAX_PTS + slot, TRASH)].add(order + 1)
    pidx = pidx[:TRASH]
    vlen = jnp.zeros((VIDS_PAD + 1,), jnp.int32).at[vid].add(
        valid.astype(jnp.int32))[:VIDS_PAD]
    vkey = jnp.zeros((VIDS_PAD + 1,), jnp.int32).at[
        jnp.where(vstart, vid, VIDS_PAD)].add(skey)[:VIDS_PAD]

    pts_flat = jnp.concatenate(
        [points.reshape(-1), jnp.zeros((NFLAT - n * 5,), jnp.float32)])
    pts8 = _sc_pad_rows(pts_flat)
    vox_flat, vc_flat, np_pad = _sc_materialize(pts8, pidx, vlen, vkey)
    voxels = vox_flat[: VIDS * MAX_PTS * NFEAT].reshape(VIDS, MAX_PTS, NFEAT)
    vcoords = vc_flat[: VIDS * 4].reshape(VIDS, 4)
    return voxels, vcoords, np_pad[:VIDS]
